# baseline (speedup 1.0000x reference)
import functools

import jax
import jax.numpy as jnp
from jax.experimental import pallas as pl
from jax.experimental.pallas import tpu as pltpu

N_NODES = 10000
HID = 64
N_CLASSES = 9


_FC_BK = 6400
_FC_STEPS = (N_NODES * HID) // _FC_BK


def _fc_body(h_ref, wfc_ref, bfc_ref, out_ref, acc_ref):
    k = pl.program_id(0)

    @pl.when(k == 0)
    def _():
        acc_ref[...] = jnp.zeros_like(acc_ref)

    acc_ref[...] += jnp.dot(h_ref[...], wfc_ref[...],
                            preferred_element_type=jnp.float32)

    @pl.when(k == _FC_STEPS - 1)
    def _():
        logits = acc_ref[...] + bfc_ref[...]
        m = jnp.max(logits, axis=-1, keepdims=True)
        z = logits - m
        lse = jnp.log(jnp.sum(jnp.exp(z), axis=-1, keepdims=True))
        out_ref[...] = z - lse


def _fc_logsoftmax(h_flat, Wfc, bfc):
    return pl.pallas_call(
        _fc_body,
        grid=(_FC_STEPS,),
        in_specs=[
            pl.BlockSpec((1, _FC_BK), lambda k: (0, k)),
            pl.BlockSpec((_FC_BK, N_CLASSES), lambda k: (k, 0)),
            pl.BlockSpec((1, N_CLASSES), lambda k: (0, 0)),
        ],
        out_specs=pl.BlockSpec((1, N_CLASSES), lambda k: (0, 0)),
        scratch_shapes=[pltpu.VMEM((1, N_CLASSES), jnp.float32)],
        out_shape=jax.ShapeDtypeStruct((1, N_CLASSES), jnp.float32),
    )(h_flat.reshape(1, -1), Wfc, bfc.reshape(1, -1))


def _gcn_conv(x, W, b, src, dst, w, n_nodes):
    h = x @ W
    deg = jax.ops.segment_sum(w, dst, num_segments=n_nodes)
    dinv = jnp.where(deg > 0, jax.lax.rsqrt(deg), 0.0)
    norm = dinv[src] * w * dinv[dst]
    msg = h[src] * norm[:, None]
    out = jax.ops.segment_sum(msg, dst, num_segments=n_nodes)
    return out + b


def kernel(x, edge_index, edge_weight, W1, b1, W2, b2, W3, b3, Wfc, bfc):
    n = x.shape[0]
    loop = jnp.arange(n)
    src = jnp.concatenate([edge_index[0], loop])
    dst = jnp.concatenate([edge_index[1], loop])
    w = jnp.concatenate([edge_weight, jnp.ones((n,), dtype=x.dtype)])
    h = jax.nn.leaky_relu(_gcn_conv(x, W1, b1, src, dst, w, n), 0.01)
    h = jax.nn.leaky_relu(_gcn_conv(h, W2, b2, src, dst, w, n), 0.01)
    h = jax.nn.leaky_relu(_gcn_conv(h, W3, b3, src, dst, w, n), 0.01)
    return _fc_logsoftmax(h, Wfc, bfc)[0]


# speedup vs baseline: 7.4078x; 7.4078x over previous
import functools

import jax
import jax.numpy as jnp
from jax import lax
from jax.experimental import pallas as pl
from jax.experimental.pallas import tpu as pltpu
from jax.experimental.pallas import tpu_sc as plsc

N_NODES = 10000
N_EDGES = 320000
HID = 64
N_CLASSES = 9

_NC = 2
_NS = 16
_NW = _NC * _NS
_EPW = N_EDGES // _NW
_K = 80
_STEPS = _EPW // _K
_NP = 10240
_RPW = _NP // _NS


def _make_prop(D):
    mesh = plsc.VectorSubcoreMesh(core_axis_name="c", subcore_axis_name="s")

    @functools.partial(
        pl.kernel,
        mesh=mesh,
        out_type=jax.ShapeDtypeStruct((_NC, _NP, D), jnp.float32),
        scratch_types=[
            pltpu.VMEM((_K,), jnp.int32),
            pltpu.VMEM((_K,), jnp.int32),
            pltpu.VMEM((_K,), jnp.float32),
            pltpu.VMEM((_K, D), jnp.float32),
            pltpu.VMEM_SHARED((_NP, D), jnp.float32),
            pltpu.SemaphoreType.DMA,
        ],
        compiler_params=pltpu.CompilerParams(
            needs_layout_passes=False, use_tc_tiling_on_sc=False),
    )
    def prop(g_hbm, src_hbm, dst_hbm, w_hbm, z_hbm, out_hbm,
             src_v, dst_v, w_v, rows_v, acc_sh, sem):
        cid = lax.axis_index("c")
        sid = lax.axis_index("s")
        wid = sid * _NC + cid
        r0 = sid * _RPW
        pltpu.sync_copy(z_hbm.at[pl.ds(r0, _RPW)], acc_sh.at[pl.ds(r0, _RPW)])
        plsc.subcore_barrier()

        base = wid * _EPW

        def body(t, carry):
            off = base + t * _K
            pltpu.sync_copy(src_hbm.at[pl.ds(off, _K)], src_v)
            pltpu.sync_copy(dst_hbm.at[pl.ds(off, _K)], dst_v)
            pltpu.sync_copy(w_hbm.at[pl.ds(off, _K)], w_v)
            pltpu.async_copy(g_hbm.at[src_v], rows_v, sem).wait()
            for j in range(_K):
                wspl = plsc.load_gather(
                    w_v, [jnp.full((16,), j, jnp.int32)])
                for d in range(D // 16):
                    sl = pl.ds(d * 16, 16)
                    rows_v[j, sl] = rows_v[j, sl] * wspl
            pltpu.sync_copy(rows_v, acc_sh.at[dst_v], add=True)
            return carry

        lax.fori_loop(0, _STEPS, body, 0)
        plsc.subcore_barrier()
        pltpu.sync_copy(acc_sh.at[pl.ds(r0, _RPW)],
                        out_hbm.at[cid, pl.ds(r0, _RPW)])

    return prop


_prop16 = _make_prop(16)
_prop64 = _make_prop(64)


def _leaky(v):
    return jnp.maximum(v, 0.01 * v)


def _k_dinv_body(a0_ref, a1_ref, x_ref, dinv_ref, g0_ref):
    deg = a0_ref[:, 0:1] + a1_ref[:, 0:1] + 1.0
    dv = lax.rsqrt(deg)
    dinv_ref[...] = dv
    lanes = lax.broadcasted_iota(jnp.int32, (_NP, 16), 1)
    g0_ref[...] = jnp.where(lanes == 0, dv * x_ref[...], 0.0)


def _k_dinv(a0, a1, x):
    return pl.pallas_call(
        _k_dinv_body,
        out_shape=(jax.ShapeDtypeStruct((_NP, 1), jnp.float32),
                   jax.ShapeDtypeStruct((_NP, 16), jnp.float32)),
    )(a0, a1, x)


def _k_l1_body(a0_ref, a1_ref, g0_ref, dinv_ref, w1_ref, b1_ref, g1_ref):
    dv = dinv_ref[...]
    p0 = dv * (a0_ref[:, 0:1] + a1_ref[:, 0:1] + g0_ref[:, 0:1])
    h1 = _leaky(p0 * w1_ref[...] + b1_ref[...])
    g1_ref[...] = dv * h1


def _k_l1(a0, a1, g0, dinv, w1p, b1p):
    return pl.pallas_call(
        _k_l1_body,
        out_shape=jax.ShapeDtypeStruct((_NP, 16), jnp.float32),
    )(a0, a1, g0, dinv, w1p, b1p)


def _k_l23_body(a0_ref, a1_ref, g1_ref, dinv_ref, w2_ref, b2_ref, w3_ref,
                g2_ref):
    dv = dinv_ref[...]
    q1 = dv * (a0_ref[...] + a1_ref[...] + g1_ref[...])
    h2 = _leaky(jnp.dot(q1, w2_ref[...],
                        preferred_element_type=jnp.float32) + b2_ref[...])
    m2 = jnp.dot(h2, w3_ref[...], preferred_element_type=jnp.float32)
    g2_ref[...] = dv * m2


def _k_l23(a0, a1, g1, dinv, w2p, b2, w3):
    return pl.pallas_call(
        _k_l23_body,
        out_shape=jax.ShapeDtypeStruct((_NP, HID), jnp.float32),
    )(a0, a1, g1, dinv, w2p, b2, w3)


_FB = 1000
_FSTEPS = N_NODES // _FB
_RCOLS = HID * N_CLASSES


def _k_fc_body(a0_ref, a1_ref, g2_ref, dinv_ref, b3_ref, r_ref, bfc_ref,
               out_ref, m_ref):
    k = pl.program_id(0)

    @pl.when(k == 0)
    def _():
        m_ref[...] = jnp.zeros_like(m_ref)

    dv = dinv_ref[...]
    q2 = dv * (a0_ref[...] + a1_ref[...] + g2_ref[...])
    h3 = _leaky(q2 + b3_ref[...])
    m_ref[...] += lax.dot_general(
        h3, r_ref[...], (((0,), (0,)), ((), ())),
        preferred_element_type=jnp.float32)

    @pl.when(k == _FSTEPS - 1)
    def _():
        m = m_ref[...]
        rows = lax.broadcasted_iota(jnp.int32, (HID, _RCOLS), 0)
        cols = lax.broadcasted_iota(jnp.int32, (HID, _RCOLS), 1)
        sel = (cols // N_CLASSES) == rows
        s = jnp.sum(jnp.where(sel, m, 0.0), axis=0, keepdims=True)
        gi = lax.broadcasted_iota(jnp.int32, (_RCOLS, N_CLASSES), 0)
        gc = lax.broadcasted_iota(jnp.int32, (_RCOLS, N_CLASSES), 1)
        gmat = ((gi % N_CLASSES) == gc).astype(jnp.float32)
        logits = jnp.dot(s, gmat,
                         preferred_element_type=jnp.float32) + bfc_ref[...]
        mx = jnp.max(logits, axis=-1, keepdims=True)
        z = logits - mx
        lse = jnp.log(jnp.sum(jnp.exp(z), axis=-1, keepdims=True))
        out_ref[...] = z - lse


def _k_fc(a0, a1, g2, dinv, b3, R, bfc):
    return pl.pallas_call(
        _k_fc_body,
        grid=(_FSTEPS,),
        in_specs=[
            pl.BlockSpec((_FB, HID), lambda k: (k, 0)),
            pl.BlockSpec((_FB, HID), lambda k: (k, 0)),
            pl.BlockSpec((_FB, HID), lambda k: (k, 0)),
            pl.BlockSpec((_FB, 1), lambda k: (k, 0)),
            pl.BlockSpec((1, HID), lambda k: (0, 0)),
            pl.BlockSpec((_FB, _RCOLS), lambda k: (k, 0)),
            pl.BlockSpec((1, N_CLASSES), lambda k: (0, 0)),
        ],
        out_specs=pl.BlockSpec((1, N_CLASSES), lambda k: (0, 0)),
        scratch_shapes=[pltpu.VMEM((HID, _RCOLS), jnp.float32)],
        out_shape=jax.ShapeDtypeStruct((1, N_CLASSES), jnp.float32),
    )(a0, a1, g2, dinv, b3, R, bfc)


def kernel(x, edge_index, edge_weight, W1, b1, W2, b2, W3, b3, Wfc, bfc):
    src = edge_index[0]
    dst = edge_index[1]
    w = edge_weight
    xp = jnp.pad(x, ((0, _NP - N_NODES), (0, 0)))
    z16 = jnp.zeros((_NP, 16), jnp.float32)
    z64 = jnp.zeros((_NP, HID), jnp.float32)
    ones16 = jnp.where(
        lax.broadcasted_iota(jnp.int32, (_NP, 16), 1) == 0, 1.0, 0.0)

    dacc = _prop16(ones16, src, dst, w, z16)
    dinv, g0 = _k_dinv(dacc[0], dacc[1], xp)

    p0 = _prop16(g0, src, dst, w, z16)
    w1p = jnp.pad(W1, ((0, 0), (0, 16 - W1.shape[1])))
    b1p = jnp.pad(b1, (0, 16 - b1.shape[0])).reshape(1, 16)
    g1 = _k_l1(p0[0], p0[1], g0, dinv, w1p, b1p)

    p1 = _prop16(g1, src, dst, w, z16)
    w2p = jnp.pad(W2, ((0, 16 - W2.shape[0]), (0, 0)))
    g2 = _k_l23(p1[0], p1[1], g1, dinv, w2p, b2.reshape(1, HID), W3)

    p2 = _prop64(g2, src, dst, w, z64)
    R = Wfc.reshape(N_NODES, _RCOLS)
    out = _k_fc(p2[0], p2[1], g2, dinv, b3.reshape(1, HID), R,
                bfc.reshape(1, N_CLASSES))
    return out[0]


# speedup vs baseline: 15.8520x; 2.1399x over previous
import functools

import jax
import jax.numpy as jnp
from jax import lax
from jax.experimental import pallas as pl
from jax.experimental.pallas import tpu as pltpu
from jax.experimental.pallas import tpu_sc as plsc

N_NODES = 10000
N_EDGES = 320000
HID = 64
N_CLASSES = 9

_NC = 2
_NS = 16
_NW = _NC * _NS
_EPW = N_EDGES // _NW
_K = 80
_STEPS = _EPW // _K
_NP = 10240
_RPW = _NP // _NS


def _make_prop(D):
    mesh = plsc.VectorSubcoreMesh(core_axis_name="c", subcore_axis_name="s")

    @functools.partial(
        pl.kernel,
        mesh=mesh,
        out_type=jax.ShapeDtypeStruct((_NC, _NP, D), jnp.float32),
        scratch_types=[
            pltpu.VMEM((_EPW,), jnp.int32),
            pltpu.VMEM((_STEPS, _K), jnp.int32),
            pltpu.VMEM((_EPW,), jnp.float32),
            pltpu.VMEM((2, _K, D), jnp.float32),
            pltpu.VMEM_SHARED((_NP, D), jnp.float32),
            pltpu.SemaphoreType.DMA,
            pltpu.SemaphoreType.DMA,
        ],
        compiler_params=pltpu.CompilerParams(
            needs_layout_passes=False, use_tc_tiling_on_sc=False),
    )
    def prop(g_hbm, src_hbm, dst_hbm, w_hbm, z_hbm, out_hbm,
             src_v, dst_v, w_v, rows_v, acc_sh, sem0, sem1):
        cid = lax.axis_index("c")
        sid = lax.axis_index("s")
        wid = sid * _NC + cid
        r0 = sid * _RPW
        pltpu.sync_copy(z_hbm.at[pl.ds(r0, _RPW)], acc_sh.at[pl.ds(r0, _RPW)])
        pltpu.sync_copy(src_hbm.at[wid], src_v)
        pltpu.sync_copy(dst_hbm.at[wid], dst_v)
        pltpu.sync_copy(w_hbm.at[wid], w_v)
        plsc.subcore_barrier()

        sems = (sem0, sem1)

        def gather(t, b, sem):
            pltpu.async_copy(g_hbm.at[src_v.at[pl.ds(t * _K, _K)]],
                             rows_v.at[b], sem)

        def wait(b, sem):
            pltpu.make_async_copy(g_hbm.at[src_v.at[pl.ds(0, _K)]],
                                  rows_v.at[b], sem).wait()

        def scale_scatter(t, b):
            for j in range(_K):
                wspl = plsc.load_gather(
                    w_v, [jnp.full((16,), t * _K + j, jnp.int32)])
                for d in range(D // 16):
                    sl = pl.ds(d * 16, 16)
                    rows_v[b, j, sl] = rows_v[b, j, sl] * wspl
            pltpu.sync_copy(rows_v.at[b], acc_sh.at[dst_v.at[t]], add=True)

        gather(0, 0, sem0)

        def body(tt, carry):
            t0 = tt * 2
            t1 = t0 + 1
            wait(0, sem0)
            gather(t1, 1, sem1)
            scale_scatter(t0, 0)
            wait(1, sem1)
            gather(t0 + 2, 0, sem0)
            scale_scatter(t1, 1)
            return carry

        lax.fori_loop(0, (_STEPS - 1) // 2, body, 0)
        wait(0, sem0)
        scale_scatter(_STEPS - 1, 0)

        plsc.subcore_barrier()
        pltpu.sync_copy(acc_sh.at[pl.ds(r0, _RPW)],
                        out_hbm.at[cid, pl.ds(r0, _RPW)])

    return prop


_prop16 = _make_prop(16)
_prop64 = _make_prop(64)


def _leaky(v):
    return jnp.maximum(v, 0.01 * v)


def _k_dinv_body(a0_ref, a1_ref, x_ref, dinv_ref, g0_ref):
    deg = a0_ref[:, 0:1] + a1_ref[:, 0:1] + 1.0
    dv = lax.rsqrt(deg)
    dinv_ref[...] = dv
    lanes = lax.broadcasted_iota(jnp.int32, (_NP, 16), 1)
    g0_ref[...] = jnp.where(lanes == 0, dv * x_ref[...], 0.0)


def _k_dinv(a0, a1, x):
    return pl.pallas_call(
        _k_dinv_body,
        out_shape=(jax.ShapeDtypeStruct((_NP, 1), jnp.float32),
                   jax.ShapeDtypeStruct((_NP, 16), jnp.float32)),
    )(a0, a1, x)


def _k_l1_body(a0_ref, a1_ref, g0_ref, dinv_ref, w1_ref, b1_ref, g1_ref):
    dv = dinv_ref[...]
    p0 = dv * (a0_ref[:, 0:1] + a1_ref[:, 0:1] + g0_ref[:, 0:1])
    h1 = _leaky(p0 * w1_ref[...] + b1_ref[...])
    g1_ref[...] = dv * h1


def _k_l1(a0, a1, g0, dinv, w1p, b1p):
    return pl.pallas_call(
        _k_l1_body,
        out_shape=jax.ShapeDtypeStruct((_NP, 16), jnp.float32),
    )(a0, a1, g0, dinv, w1p, b1p)


def _k_l23_body(a0_ref, a1_ref, g1_ref, dinv_ref, w2_ref, b2_ref, w3_ref,
                g2_ref):
    dv = dinv_ref[...]
    q1 = dv * (a0_ref[...] + a1_ref[...] + g1_ref[...])
    h2 = _leaky(jnp.dot(q1, w2_ref[...],
                        preferred_element_type=jnp.float32) + b2_ref[...])
    m2 = jnp.dot(h2, w3_ref[...], preferred_element_type=jnp.float32)
    g2_ref[...] = dv * m2


def _k_l23(a0, a1, g1, dinv, w2p, b2, w3):
    return pl.pallas_call(
        _k_l23_body,
        out_shape=jax.ShapeDtypeStruct((_NP, HID), jnp.float32),
    )(a0, a1, g1, dinv, w2p, b2, w3)


_FB = 1000
_FSTEPS = N_NODES // _FB
_RCOLS = HID * N_CLASSES


def _k_fc_body(a0_ref, a1_ref, g2_ref, dinv_ref, b3_ref, r_ref, bfc_ref,
               out_ref, m_ref):
    k = pl.program_id(0)

    @pl.when(k == 0)
    def _():
        m_ref[...] = jnp.zeros_like(m_ref)

    dv = dinv_ref[...]
    q2 = dv * (a0_ref[...] + a1_ref[...] + g2_ref[...])
    h3 = _leaky(q2 + b3_ref[...])
    m_ref[...] += lax.dot_general(
        h3, r_ref[...], (((0,), (0,)), ((), ())),
        preferred_element_type=jnp.float32)

    @pl.when(k == _FSTEPS - 1)
    def _():
        m = m_ref[...]
        rows = lax.broadcasted_iota(jnp.int32, (HID, _RCOLS), 0)
        cols = lax.broadcasted_iota(jnp.int32, (HID, _RCOLS), 1)
        sel = (cols // N_CLASSES) == rows
        s = jnp.sum(jnp.where(sel, m, 0.0), axis=0, keepdims=True)
        gi = lax.broadcasted_iota(jnp.int32, (_RCOLS, N_CLASSES), 0)
        gc = lax.broadcasted_iota(jnp.int32, (_RCOLS, N_CLASSES), 1)
        gmat = ((gi % N_CLASSES) == gc).astype(jnp.float32)
        logits = jnp.dot(s, gmat,
                         preferred_element_type=jnp.float32) + bfc_ref[...]
        mx = jnp.max(logits, axis=-1, keepdims=True)
        z = logits - mx
        lse = jnp.log(jnp.sum(jnp.exp(z), axis=-1, keepdims=True))
        out_ref[...] = z - lse


def _k_fc(a0, a1, g2, dinv, b3, R, bfc):
    return pl.pallas_call(
        _k_fc_body,
        grid=(_FSTEPS,),
        in_specs=[
            pl.BlockSpec((_FB, HID), lambda k: (k, 0)),
            pl.BlockSpec((_FB, HID), lambda k: (k, 0)),
            pl.BlockSpec((_FB, HID), lambda k: (k, 0)),
            pl.BlockSpec((_FB, 1), lambda k: (k, 0)),
            pl.BlockSpec((1, HID), lambda k: (0, 0)),
            pl.BlockSpec((_FB, _RCOLS), lambda k: (k, 0)),
            pl.BlockSpec((1, N_CLASSES), lambda k: (0, 0)),
        ],
        out_specs=pl.BlockSpec((1, N_CLASSES), lambda k: (0, 0)),
        scratch_shapes=[pltpu.VMEM((HID, _RCOLS), jnp.float32)],
        out_shape=jax.ShapeDtypeStruct((1, N_CLASSES), jnp.float32),
    )(a0, a1, g2, dinv, b3, R, bfc)


def kernel(x, edge_index, edge_weight, W1, b1, W2, b2, W3, b3, Wfc, bfc):
    src = edge_index[0].reshape(_NW, _EPW)
    dst = edge_index[1].reshape(_NW, _STEPS, _K)
    w = edge_weight.reshape(_NW, _EPW)
    xp = jnp.pad(x, ((0, _NP - N_NODES), (0, 0)))
    z16 = jnp.zeros((_NP, 16), jnp.float32)
    z64 = jnp.zeros((_NP, HID), jnp.float32)
    ones16 = jnp.where(
        lax.broadcasted_iota(jnp.int32, (_NP, 16), 1) == 0, 1.0, 0.0)

    dacc = _prop16(ones16, src, dst, w, z16)
    dinv, g0 = _k_dinv(dacc[0], dacc[1], xp)

    p0 = _prop16(g0, src, dst, w, z16)
    w1p = jnp.pad(W1, ((0, 0), (0, 16 - W1.shape[1])))
    b1p = jnp.pad(b1, (0, 16 - b1.shape[0])).reshape(1, 16)
    g1 = _k_l1(p0[0], p0[1], g0, dinv, w1p, b1p)

    p1 = _prop16(g1, src, dst, w, z16)
    w2p = jnp.pad(W2, ((0, 16 - W2.shape[0]), (0, 0)))
    g2 = _k_l23(p1[0], p1[1], g1, dinv, w2p, b2.reshape(1, HID), W3)

    p2 = _prop64(g2, src, dst, w, z64)
    R = Wfc.reshape(N_NODES, _RCOLS)
    out = _k_fc(p2[0], p2[1], g2, dinv, b3.reshape(1, HID), R,
                bfc.reshape(1, N_CLASSES))
    return out[0]


# speedup vs baseline: 16.5327x; 1.0429x over previous
import functools

import jax
import jax.numpy as jnp
from jax import lax
from jax.experimental import pallas as pl
from jax.experimental.pallas import tpu as pltpu
from jax.experimental.pallas import tpu_sc as plsc

N_NODES = 10000
N_EDGES = 320000
HID = 64
N_CLASSES = 9

_NC = 2
_NS = 16
_NW = _NC * _NS
_EPW = N_EDGES // _NW
_K = 80
_STEPS = _EPW // _K
_NP = 10240
_RPW = _NP // _NS


def _make_prop(D):
    mesh = plsc.VectorSubcoreMesh(core_axis_name="c", subcore_axis_name="s")

    @functools.partial(
        pl.kernel,
        mesh=mesh,
        out_type=jax.ShapeDtypeStruct((_NC, _NP, D), jnp.float32),
        scratch_types=[
            pltpu.VMEM((_EPW,), jnp.int32),
            pltpu.VMEM((_STEPS, _K), jnp.int32),
            pltpu.VMEM((_EPW,), jnp.float32),
            pltpu.VMEM((2, _K, D), jnp.float32),
            pltpu.VMEM_SHARED((_NP, D), jnp.float32),
            pltpu.SemaphoreType.DMA,
            pltpu.SemaphoreType.DMA,
        ],
        compiler_params=pltpu.CompilerParams(
            needs_layout_passes=False, use_tc_tiling_on_sc=False),
    )
    def prop(g_hbm, src_hbm, dst_hbm, w_hbm, z_hbm, out_hbm,
             src_v, dst_v, w_v, rows_v, acc_sh, sem0, sem1):
        cid = lax.axis_index("c")
        sid = lax.axis_index("s")
        wid = sid * _NC + cid
        r0 = sid * _RPW
        pltpu.sync_copy(z_hbm.at[pl.ds(r0, _RPW)], acc_sh.at[pl.ds(r0, _RPW)])
        pltpu.sync_copy(src_hbm.at[wid], src_v)
        pltpu.sync_copy(dst_hbm.at[wid], dst_v)
        pltpu.sync_copy(w_hbm.at[wid], w_v)
        plsc.subcore_barrier()

        sems = (sem0, sem1)

        def gather(t, b, sem):
            pltpu.async_copy(g_hbm.at[src_v.at[pl.ds(t * _K, _K)]],
                             rows_v.at[b], sem)

        def wait(b, sem):
            pltpu.make_async_copy(g_hbm.at[src_v.at[pl.ds(0, _K)]],
                                  rows_v.at[b], sem).wait()

        def scale_scatter(t, b):
            for j in range(_K):
                wspl = plsc.load_gather(
                    w_v, [jnp.full((16,), t * _K + j, jnp.int32)])
                for d in range(D // 16):
                    sl = pl.ds(d * 16, 16)
                    rows_v[b, j, sl] = rows_v[b, j, sl] * wspl
            pltpu.sync_copy(rows_v.at[b], acc_sh.at[dst_v.at[t]], add=True)

        gather(0, 0, sem0)

        def body(tt, carry):
            t0 = tt * 2
            t1 = t0 + 1
            wait(0, sem0)
            gather(t1, 1, sem1)
            scale_scatter(t0, 0)
            wait(1, sem1)
            gather(t0 + 2, 0, sem0)
            scale_scatter(t1, 1)
            return carry

        lax.fori_loop(0, (_STEPS - 1) // 2, body, 0)
        wait(0, sem0)
        scale_scatter(_STEPS - 1, 0)

        plsc.subcore_barrier()
        pltpu.sync_copy(acc_sh.at[pl.ds(r0, _RPW)],
                        out_hbm.at[cid, pl.ds(r0, _RPW)])

    return prop


_prop16 = _make_prop(16)
_prop64 = _make_prop(64)


def _make_deg():
    mesh = plsc.VectorSubcoreMesh(core_axis_name="c", subcore_axis_name="s")

    @functools.partial(
        pl.kernel,
        mesh=mesh,
        out_type=jax.ShapeDtypeStruct((_NC, _NP, 16), jnp.float32),
        scratch_types=[
            pltpu.VMEM((_STEPS, _K), jnp.int32),
            pltpu.VMEM((_EPW,), jnp.float32),
            pltpu.VMEM((_K, 16), jnp.float32),
            pltpu.VMEM_SHARED((_NP, 16), jnp.float32),
        ],
        compiler_params=pltpu.CompilerParams(
            needs_layout_passes=False, use_tc_tiling_on_sc=False),
    )
    def deg(dst_hbm, w_hbm, z_hbm, out_hbm, dst_v, w_v, rows_v, acc_sh):
        cid = lax.axis_index("c")
        sid = lax.axis_index("s")
        wid = sid * _NC + cid
        r0 = sid * _RPW
        pltpu.sync_copy(z_hbm.at[pl.ds(r0, _RPW)], acc_sh.at[pl.ds(r0, _RPW)])
        pltpu.sync_copy(dst_hbm.at[wid], dst_v)
        pltpu.sync_copy(w_hbm.at[wid], w_v)
        plsc.subcore_barrier()

        col0 = (lax.iota(jnp.int32, 16) == 0).astype(jnp.float32)

        def body(t, carry):
            for j in range(_K):
                wspl = plsc.load_gather(
                    w_v, [jnp.full((16,), t * _K + j, jnp.int32)])
                rows_v[j, pl.ds(0, 16)] = wspl * col0
            pltpu.sync_copy(rows_v, acc_sh.at[dst_v.at[t]], add=True)
            return carry

        lax.fori_loop(0, _STEPS, body, 0)
        plsc.subcore_barrier()
        pltpu.sync_copy(acc_sh.at[pl.ds(r0, _RPW)],
                        out_hbm.at[cid, pl.ds(r0, _RPW)])

    return deg


_degpass = _make_deg()


def _leaky(v):
    return jnp.maximum(v, 0.01 * v)


def _k_dinv_body(a0_ref, a1_ref, x_ref, dinv_ref, g0_ref):
    deg = a0_ref[:, 0:1] + a1_ref[:, 0:1] + 1.0
    dv = lax.rsqrt(deg)
    dinv_ref[...] = dv
    lanes = lax.broadcasted_iota(jnp.int32, (_NP, 16), 1)
    g0_ref[...] = jnp.where(lanes == 0, dv * x_ref[...], 0.0)


def _k_dinv(a0, a1, x):
    return pl.pallas_call(
        _k_dinv_body,
        out_shape=(jax.ShapeDtypeStruct((_NP, 1), jnp.float32),
                   jax.ShapeDtypeStruct((_NP, 16), jnp.float32)),
    )(a0, a1, x)


def _k_l1_body(a0_ref, a1_ref, g0_ref, dinv_ref, w1_ref, b1_ref, g1_ref):
    dv = dinv_ref[...]
    p0 = dv * (a0_ref[:, 0:1] + a1_ref[:, 0:1] + g0_ref[:, 0:1])
    h1 = _leaky(p0 * w1_ref[...] + b1_ref[...])
    g1_ref[...] = dv * h1


def _k_l1(a0, a1, g0, dinv, w1p, b1p):
    return pl.pallas_call(
        _k_l1_body,
        out_shape=jax.ShapeDtypeStruct((_NP, 16), jnp.float32),
    )(a0, a1, g0, dinv, w1p, b1p)


def _k_l23_body(a0_ref, a1_ref, g1_ref, dinv_ref, w2_ref, b2_ref, w3_ref,
                g2_ref):
    dv = dinv_ref[...]
    q1 = dv * (a0_ref[...] + a1_ref[...] + g1_ref[...])
    h2 = _leaky(jnp.dot(q1, w2_ref[...],
                        preferred_element_type=jnp.float32) + b2_ref[...])
    m2 = jnp.dot(h2, w3_ref[...], preferred_element_type=jnp.float32)
    g2_ref[...] = dv * m2


def _k_l23(a0, a1, g1, dinv, w2p, b2, w3):
    return pl.pallas_call(
        _k_l23_body,
        out_shape=jax.ShapeDtypeStruct((_NP, HID), jnp.float32),
    )(a0, a1, g1, dinv, w2p, b2, w3)


_FB = 1000
_FSTEPS = N_NODES // _FB
_RCOLS = HID * N_CLASSES


def _k_fc_body(a0_ref, a1_ref, g2_ref, dinv_ref, b3_ref, r_ref, bfc_ref,
               out_ref, m_ref):
    k = pl.program_id(0)

    @pl.when(k == 0)
    def _():
        m_ref[...] = jnp.zeros_like(m_ref)

    dv = dinv_ref[...]
    q2 = dv * (a0_ref[...] + a1_ref[...] + g2_ref[...])
    h3 = _leaky(q2 + b3_ref[...])
    m_ref[...] += lax.dot_general(
        h3, r_ref[...], (((0,), (0,)), ((), ())),
        preferred_element_type=jnp.float32)

    @pl.when(k == _FSTEPS - 1)
    def _():
        m = m_ref[...]
        rows = lax.broadcasted_iota(jnp.int32, (HID, _RCOLS), 0)
        cols = lax.broadcasted_iota(jnp.int32, (HID, _RCOLS), 1)
        sel = (cols // N_CLASSES) == rows
        s = jnp.sum(jnp.where(sel, m, 0.0), axis=0, keepdims=True)
        gi = lax.broadcasted_iota(jnp.int32, (_RCOLS, N_CLASSES), 0)
        gc = lax.broadcasted_iota(jnp.int32, (_RCOLS, N_CLASSES), 1)
        gmat = ((gi % N_CLASSES) == gc).astype(jnp.float32)
        logits = jnp.dot(s, gmat,
                         preferred_element_type=jnp.float32) + bfc_ref[...]
        mx = jnp.max(logits, axis=-1, keepdims=True)
        z = logits - mx
        lse = jnp.log(jnp.sum(jnp.exp(z), axis=-1, keepdims=True))
        out_ref[...] = z - lse


def _k_fc(a0, a1, g2, dinv, b3, R, bfc):
    return pl.pallas_call(
        _k_fc_body,
        grid=(_FSTEPS,),
        in_specs=[
            pl.BlockSpec((_FB, HID), lambda k: (k, 0)),
            pl.BlockSpec((_FB, HID), lambda k: (k, 0)),
            pl.BlockSpec((_FB, HID), lambda k: (k, 0)),
            pl.BlockSpec((_FB, 1), lambda k: (k, 0)),
            pl.BlockSpec((1, HID), lambda k: (0, 0)),
            pl.BlockSpec((_FB, _RCOLS), lambda k: (k, 0)),
            pl.BlockSpec((1, N_CLASSES), lambda k: (0, 0)),
        ],
        out_specs=pl.BlockSpec((1, N_CLASSES), lambda k: (0, 0)),
        scratch_shapes=[pltpu.VMEM((HID, _RCOLS), jnp.float32)],
        out_shape=jax.ShapeDtypeStruct((1, N_CLASSES), jnp.float32),
    )(a0, a1, g2, dinv, b3, R, bfc)


def kernel(x, edge_index, edge_weight, W1, b1, W2, b2, W3, b3, Wfc, bfc):
    src = edge_index[0].reshape(_NW, _EPW)
    dst = edge_index[1].reshape(_NW, _STEPS, _K)
    w = edge_weight.reshape(_NW, _EPW)
    xp = jnp.pad(x, ((0, _NP - N_NODES), (0, 0)))
    z16 = jnp.zeros((_NP, 16), jnp.float32)
    z64 = jnp.zeros((_NP, HID), jnp.float32)

    dacc = _degpass(dst, w, z16)
    dinv, g0 = _k_dinv(dacc[0], dacc[1], xp)

    p0 = _prop16(g0, src, dst, w, z16)
    w1p = jnp.pad(W1, ((0, 0), (0, 16 - W1.shape[1])))
    b1p = jnp.pad(b1, (0, 16 - b1.shape[0])).reshape(1, 16)
    g1 = _k_l1(p0[0], p0[1], g0, dinv, w1p, b1p)

    p1 = _prop16(g1, src, dst, w, z16)
    w2p = jnp.pad(W2, ((0, 16 - W2.shape[0]), (0, 0)))
    g2 = _k_l23(p1[0], p1[1], g1, dinv, w2p, b2.reshape(1, HID), W3)

    p2 = _prop64(g2, src, dst, w, z64)
    R = Wfc.reshape(N_NODES, _RCOLS)
    out = _k_fc(p2[0], p2[1], g2, dinv, b3.reshape(1, HID), R,
                bfc.reshape(1, N_CLASSES))
    return out[0]
